# Initial kernel scaffold; baseline (speedup 1.0000x reference)
import jax
import jax.numpy as jnp
from jax.experimental import pallas as pl


def kernel(x, ln1_w, ln1_b, in_proj_w, in_proj_b, out_proj_w, out_proj_b, ln2_w, ln2_b, gate_w, gate_b, W1, B1, W2, B2):
    raise NotImplementedError("write your pallas kernel here")


# speedup vs baseline: 2.7874x; 2.7874x over previous
import functools

import jax
import jax.numpy as jnp
from jax.experimental import pallas as pl
from jax.experimental.pallas import tpu as pltpu

S, D, H, E, TOPK = 2048, 768, 12, 16, 2
DFF = 4 * D
DH = D // H
TB = 128
PAD = S * TOPK + E * TB
NT = PAD // TB


def _k1_body(x_ref, lw_ref, lb_ref, w_ref, b_ref, qkv_ref):
    x = x_ref[...]
    mu = jnp.mean(x, axis=-1, keepdims=True)
    var = jnp.mean((x - mu) ** 2, axis=-1, keepdims=True)
    h = (x - mu) * jax.lax.rsqrt(var + 1e-5) * lw_ref[...] + lb_ref[...]
    qkv_ref[...] = jax.lax.dot_general(
        h, w_ref[...], (((1,), (1,)), ((), ())),
        preferred_element_type=jnp.float32) + b_ref[...]


def _qkv_proj(x, ln1_w, ln1_b, w, b):
    bs = 256
    return pl.pallas_call(
        _k1_body,
        grid=(S // bs,),
        in_specs=[
            pl.BlockSpec((bs, D), lambda i: (i, 0)),
            pl.BlockSpec((1, D), lambda i: (0, 0)),
            pl.BlockSpec((1, D), lambda i: (0, 0)),
            pl.BlockSpec((3 * D, D), lambda i: (0, 0)),
            pl.BlockSpec((1, 3 * D), lambda i: (0, 0)),
        ],
        out_specs=pl.BlockSpec((bs, 3 * D), lambda i: (i, 0)),
        out_shape=jax.ShapeDtypeStruct((S, 3 * D), jnp.float32),
    )(x, ln1_w.reshape(1, D), ln1_b.reshape(1, D), w, b.reshape(1, 3 * D))


def _k2_body(q_ref, k_ref, v_ref, o_ref):
    scale = 1.0 / (DH ** 0.5)
    outs = []
    for sub in range(2):
        q = q_ref[:, sub * DH:(sub + 1) * DH]
        k = k_ref[:, sub * DH:(sub + 1) * DH]
        v = v_ref[:, sub * DH:(sub + 1) * DH]
        s = jax.lax.dot_general(q, k, (((1,), (1,)), ((), ())),
                                preferred_element_type=jnp.float32) * scale
        m = jnp.max(s, axis=-1, keepdims=True)
        p = jnp.exp(s - m)
        l = jnp.sum(p, axis=-1, keepdims=True)
        o = jax.lax.dot_general(p, v, (((1,), (0,)), ((), ())),
                                preferred_element_type=jnp.float32)
        outs.append(o / l)
    o_ref[...] = jnp.concatenate(outs, axis=1)


def _attention(qkv):
    qb = 256
    return pl.pallas_call(
        _k2_body,
        grid=(H // 2, S // qb),
        in_specs=[
            pl.BlockSpec((qb, 2 * DH), lambda p, i: (i, p)),
            pl.BlockSpec((S, 2 * DH), lambda p, i: (0, 6 + p)),
            pl.BlockSpec((S, 2 * DH), lambda p, i: (0, 12 + p)),
        ],
        out_specs=pl.BlockSpec((qb, 2 * DH), lambda p, i: (i, p)),
        out_shape=jax.ShapeDtypeStruct((S, D), jnp.float32),
    )(qkv, qkv, qkv)


def _k3_body(o_ref, x_ref, w_ref, b_ref, lw_ref, lb_ref, gw_ref, gb_ref,
             x1_ref, f_ref, sc_ref):
    x1 = x_ref[...] + jax.lax.dot_general(
        o_ref[...], w_ref[...], (((1,), (1,)), ((), ())),
        preferred_element_type=jnp.float32) + b_ref[...]
    x1_ref[...] = x1
    mu = jnp.mean(x1, axis=-1, keepdims=True)
    var = jnp.mean((x1 - mu) ** 2, axis=-1, keepdims=True)
    f = (x1 - mu) * jax.lax.rsqrt(var + 1e-5) * lw_ref[...] + lb_ref[...]
    f_ref[...] = f
    sc_ref[...] = jax.lax.dot_general(
        f, gw_ref[...], (((1,), (1,)), ((), ())),
        preferred_element_type=jnp.float32) + gb_ref[...]


def _post_attn(o, x, w, b, ln2_w, ln2_b, gate_w, gate_b):
    bs = 256
    return pl.pallas_call(
        _k3_body,
        grid=(S // bs,),
        in_specs=[
            pl.BlockSpec((bs, D), lambda i: (i, 0)),
            pl.BlockSpec((bs, D), lambda i: (i, 0)),
            pl.BlockSpec((D, D), lambda i: (0, 0)),
            pl.BlockSpec((1, D), lambda i: (0, 0)),
            pl.BlockSpec((1, D), lambda i: (0, 0)),
            pl.BlockSpec((1, D), lambda i: (0, 0)),
            pl.BlockSpec((E, D), lambda i: (0, 0)),
            pl.BlockSpec((1, E), lambda i: (0, 0)),
        ],
        out_specs=[
            pl.BlockSpec((bs, D), lambda i: (i, 0)),
            pl.BlockSpec((bs, D), lambda i: (i, 0)),
            pl.BlockSpec((bs, E), lambda i: (i, 0)),
        ],
        out_shape=[
            jax.ShapeDtypeStruct((S, D), jnp.float32),
            jax.ShapeDtypeStruct((S, D), jnp.float32),
            jax.ShapeDtypeStruct((S, E), jnp.float32),
        ],
    )(o, x, w, b.reshape(1, D), ln2_w.reshape(1, D), ln2_b.reshape(1, D),
      gate_w, gate_b.reshape(1, E))


def _k6_body(te_ref, f_ref, w1_ref, b1_ref, w2_ref, b2_ref, ws_ref, y_ref):
    a = jax.lax.dot_general(
        f_ref[...], w1_ref[0], (((1,), (1,)), ((), ())),
        preferred_element_type=jnp.float32) + b1_ref[0]
    g = 0.5 * a * (1.0 + jax.lax.erf(a * (2.0 ** -0.5)))
    y = jax.lax.dot_general(
        g, w2_ref[0], (((1,), (1,)), ((), ())),
        preferred_element_type=jnp.float32) + b2_ref[0]
    y_ref[...] = y * ws_ref[0, 0][:, None]


def _grouped_ffn(tile_expert, fsorted, W1, B1, W2, B2, wsorted):
    grid_spec = pltpu.PrefetchScalarGridSpec(
        num_scalar_prefetch=1,
        grid=(NT,),
        in_specs=[
            pl.BlockSpec((TB, D), lambda t, te: (t, 0)),
            pl.BlockSpec((1, DFF, D), lambda t, te: (te[t], 0, 0)),
            pl.BlockSpec((1, 1, DFF), lambda t, te: (te[t], 0, 0)),
            pl.BlockSpec((1, D, DFF), lambda t, te: (te[t], 0, 0)),
            pl.BlockSpec((1, 1, D), lambda t, te: (te[t], 0, 0)),
            pl.BlockSpec((1, 1, TB), lambda t, te: (t, 0, 0)),
        ],
        out_specs=pl.BlockSpec((TB, D), lambda t, te: (t, 0)),
    )
    return pl.pallas_call(
        _k6_body,
        grid_spec=grid_spec,
        out_shape=jax.ShapeDtypeStruct((PAD, D), jnp.float32),
    )(tile_expert, fsorted, W1, B1.reshape(E, 1, DFF), W2,
      B2.reshape(E, 1, D), wsorted.reshape(NT, 1, TB))


def _route(scores):
    top_s, top_i = jax.lax.top_k(scores, TOPK)
    w = jax.nn.softmax(top_s, axis=-1)
    eflat = top_i.reshape(-1)
    wflat = w.reshape(-1)
    perm = jnp.argsort(eflat)
    sorted_e = eflat[perm]
    counts = jnp.bincount(eflat, length=E)
    raw_off = jnp.concatenate([jnp.zeros((1,), jnp.int32),
                               jnp.cumsum(counts)[:-1].astype(jnp.int32)])
    pcounts = ((counts + TB - 1) // TB) * TB
    poff = jnp.concatenate([jnp.zeros((1,), jnp.int32),
                            jnp.cumsum(pcounts)[:-1].astype(jnp.int32)])
    rank = jnp.arange(S * TOPK, dtype=jnp.int32) - raw_off[sorted_e]
    pos_sorted = poff[sorted_e] + rank
    pos = jnp.zeros((S * TOPK,), jnp.int32).at[perm].set(pos_sorted)
    gather_ids = jnp.zeros((PAD,), jnp.int32).at[pos_sorted].set(perm // TOPK)
    wsorted = jnp.zeros((PAD,), jnp.float32).at[pos_sorted].set(wflat[perm])
    pend = poff + pcounts
    tile_expert = jnp.searchsorted(
        pend, jnp.arange(NT, dtype=jnp.int32) * TB, side='right'
    ).astype(jnp.int32)
    tile_expert = jnp.minimum(tile_expert, E - 1)
    return pos.reshape(S, TOPK), gather_ids, wsorted, tile_expert


def kernel(x, ln1_w, ln1_b, in_proj_w, in_proj_b, out_proj_w, out_proj_b,
           ln2_w, ln2_b, gate_w, gate_b, W1, B1, W2, B2):
    x2 = x.reshape(S, D)
    qkv = _qkv_proj(x2, ln1_w, ln1_b, in_proj_w, in_proj_b)
    o = _attention(qkv)
    x1, f, scores = _post_attn(o, x2, out_proj_w, out_proj_b,
                               ln2_w, ln2_b, gate_w, gate_b)
    pos, gather_ids, wsorted, tile_expert = _route(scores)
    fsorted = f[gather_ids]
    y = _grouped_ffn(tile_expert, fsorted, W1, B1, W2, B2, wsorted)
    out = x1 + y[pos[:, 0]] + y[pos[:, 1]]
    return out.reshape(1, S, D)


# speedup vs baseline: 3.2771x; 1.1757x over previous
import functools

import jax
import jax.numpy as jnp
from jax import lax
from jax.experimental import pallas as pl
from jax.experimental.pallas import tpu as pltpu
from jax.experimental.pallas import tpu_sc as plsc

S, D, H, E, TOPK = 2048, 768, 12, 16, 2
DFF = 4 * D
DH = D // H
TB = 128
PAD = S * TOPK + E * TB
NT = PAD // TB


def _k1_body(x_ref, lw_ref, lb_ref, w_ref, b_ref, qkv_ref):
    x = x_ref[...]
    mu = jnp.mean(x, axis=-1, keepdims=True)
    var = jnp.mean((x - mu) ** 2, axis=-1, keepdims=True)
    h = (x - mu) / jnp.sqrt(var + 1e-5) * lw_ref[...] + lb_ref[...]
    qkv_ref[...] = jax.lax.dot_general(
        h, w_ref[...], (((1,), (1,)), ((), ())),
        preferred_element_type=jnp.float32) + b_ref[...]


def _qkv_proj(x, ln1_w, ln1_b, w, b):
    bs = 256
    return pl.pallas_call(
        _k1_body,
        grid=(S // bs,),
        in_specs=[
            pl.BlockSpec((bs, D), lambda i: (i, 0)),
            pl.BlockSpec((1, D), lambda i: (0, 0)),
            pl.BlockSpec((1, D), lambda i: (0, 0)),
            pl.BlockSpec((3 * D, D), lambda i: (0, 0)),
            pl.BlockSpec((1, 3 * D), lambda i: (0, 0)),
        ],
        out_specs=pl.BlockSpec((bs, 3 * D), lambda i: (i, 0)),
        out_shape=jax.ShapeDtypeStruct((S, 3 * D), jnp.float32),
    )(x, ln1_w.reshape(1, D), ln1_b.reshape(1, D), w, b.reshape(1, 3 * D))


def _k2_body(q_ref, k_ref, v_ref, o_ref):
    scale = 1.0 / (DH ** 0.5)
    outs = []
    for sub in range(2):
        q = q_ref[:, sub * DH:(sub + 1) * DH]
        k = k_ref[:, sub * DH:(sub + 1) * DH]
        v = v_ref[:, sub * DH:(sub + 1) * DH]
        s = jax.lax.dot_general(q, k, (((1,), (1,)), ((), ())),
                                                        preferred_element_type=jnp.float32) * scale
        m = jnp.max(s, axis=-1, keepdims=True)
        p = jnp.exp(s - m)
        l = jnp.sum(p, axis=-1, keepdims=True)
        o = jax.lax.dot_general(p, v, (((1,), (0,)), ((), ())),
                                                        preferred_element_type=jnp.float32)
        outs.append(o / l)
    o_ref[...] = jnp.concatenate(outs, axis=1)


def _attention(qkv):
    qb = 256
    return pl.pallas_call(
        _k2_body,
        grid=(H // 2, S // qb),
        in_specs=[
            pl.BlockSpec((qb, 2 * DH), lambda p, i: (i, p)),
            pl.BlockSpec((S, 2 * DH), lambda p, i: (0, 6 + p)),
            pl.BlockSpec((S, 2 * DH), lambda p, i: (0, 12 + p)),
        ],
        out_specs=pl.BlockSpec((qb, 2 * DH), lambda p, i: (i, p)),
        out_shape=jax.ShapeDtypeStruct((S, D), jnp.float32),
    )(qkv, qkv, qkv)


def _k3_body(o_ref, x_ref, w_ref, b_ref, lw_ref, lb_ref, gw_ref, gb_ref,
             x1_ref, f_ref, sc_ref):
    x1 = x_ref[...] + jax.lax.dot_general(
        o_ref[...], w_ref[...], (((1,), (1,)), ((), ())),
        preferred_element_type=jnp.float32) + b_ref[...]
    x1_ref[...] = x1
    mu = jnp.mean(x1, axis=-1, keepdims=True)
    var = jnp.mean((x1 - mu) ** 2, axis=-1, keepdims=True)
    f = (x1 - mu) / jnp.sqrt(var + 1e-5) * lw_ref[...] + lb_ref[...]
    f_ref[...] = f
    sc_ref[...] = jax.lax.dot_general(
        f, gw_ref[...], (((1,), (1,)), ((), ())),
        preferred_element_type=jnp.float32) + gb_ref[...]


def _post_attn(o, x, w, b, ln2_w, ln2_b, gate_w, gate_b):
    bs = 256
    return pl.pallas_call(
        _k3_body,
        grid=(S // bs,),
        in_specs=[
            pl.BlockSpec((bs, D), lambda i: (i, 0)),
            pl.BlockSpec((bs, D), lambda i: (i, 0)),
            pl.BlockSpec((D, D), lambda i: (0, 0)),
            pl.BlockSpec((1, D), lambda i: (0, 0)),
            pl.BlockSpec((1, D), lambda i: (0, 0)),
            pl.BlockSpec((1, D), lambda i: (0, 0)),
            pl.BlockSpec((E, D), lambda i: (0, 0)),
            pl.BlockSpec((1, E), lambda i: (0, 0)),
        ],
        out_specs=[
            pl.BlockSpec((bs, D), lambda i: (i, 0)),
            pl.BlockSpec((bs, D), lambda i: (i, 0)),
            pl.BlockSpec((bs, E), lambda i: (i, 0)),
        ],
        out_shape=[
            jax.ShapeDtypeStruct((S, D), jnp.float32),
            jax.ShapeDtypeStruct((S, D), jnp.float32),
            jax.ShapeDtypeStruct((S, E), jnp.float32),
        ],
    )(o, x, w, b.reshape(1, D), ln2_w.reshape(1, D), ln2_b.reshape(1, D),
      gate_w, gate_b.reshape(1, E))


def _k6_body(te_ref, f_ref, w1_ref, b1_ref, w2_ref, b2_ref, y_ref,
             w1b, w2b):
    t = pl.program_id(0)
    changed = (t == 0) | (te_ref[t] != te_ref[jnp.maximum(t - 1, 0)])

    @pl.when(changed)
    def _cast():
        w1b[...] = w1_ref[0].astype(jnp.bfloat16)
        w2b[...] = w2_ref[0].astype(jnp.bfloat16)

    a = jax.lax.dot_general(
        f_ref[...].astype(jnp.bfloat16), w1b[...], (((1,), (1,)), ((), ())),
        preferred_element_type=jnp.float32) + b1_ref[0]
    g = 0.5 * a * (1.0 + jax.lax.erf(a * (2.0 ** -0.5)))
    y = jax.lax.dot_general(
        g.astype(jnp.bfloat16), w2b[...], (((1,), (1,)), ((), ())),
        preferred_element_type=jnp.float32) + b2_ref[0]
    y_ref[...] = y


def _grouped_ffn(tile_expert, fsorted, W1, B1, W2, B2):
    grid_spec = pltpu.PrefetchScalarGridSpec(
        num_scalar_prefetch=1,
        grid=(NT,),
        in_specs=[
            pl.BlockSpec((TB, D), lambda t, te: (t, 0)),
            pl.BlockSpec((1, DFF, D), lambda t, te: (te[t], 0, 0)),
            pl.BlockSpec((1, 1, DFF), lambda t, te: (te[t], 0, 0)),
            pl.BlockSpec((1, D, DFF), lambda t, te: (te[t], 0, 0)),
            pl.BlockSpec((1, 1, D), lambda t, te: (te[t], 0, 0)),
        ],
        out_specs=pl.BlockSpec((TB, D), lambda t, te: (t, 0)),
        scratch_shapes=[
            pltpu.VMEM((DFF, D), jnp.bfloat16),
            pltpu.VMEM((D, DFF), jnp.bfloat16),
        ],
    )
    return pl.pallas_call(
        _k6_body,
        grid_spec=grid_spec,
        out_shape=jax.ShapeDtypeStruct((PAD, D), jnp.float32),
    )(tile_expert, fsorted, W1, B1.reshape(E, 1, DFF), W2,
      B2.reshape(E, 1, D))


TPA = S // 16
TPC = S // 32
CCH = 32

_mesh_a = plsc.VectorSubcoreMesh(core_axis_name="c", subcore_axis_name="s",
                                 num_cores=1)
_mesh_c = plsc.VectorSubcoreMesh(core_axis_name="c", subcore_axis_name="s")


@functools.partial(
    pl.kernel, mesh=_mesh_a,
    compiler_params=pltpu.CompilerParams(needs_layout_passes=False),
    out_type=[
        jax.ShapeDtypeStruct((S * 16,), jnp.int32),
        jax.ShapeDtypeStruct((S * 16,), jnp.float32),
        jax.ShapeDtypeStruct((PAD, D), jnp.float32),
        jax.ShapeDtypeStruct((64,), jnp.int32),
    ],
    scratch_types=[
        pltpu.VMEM((TPA * 16,), jnp.float32),
        pltpu.VMEM((TPA, D), jnp.float32),
        pltpu.VMEM((TPA * 16,), jnp.int32),
        pltpu.VMEM((TPA * 16,), jnp.float32),
        pltpu.VMEM((TPA * 16,), jnp.int32),
        pltpu.VMEM((TPA,), jnp.int32),
        pltpu.VMEM((TPA,), jnp.int32),
        pltpu.VMEM((16,), jnp.int32),
        pltpu.VMEM((256,), jnp.int32),
        pltpu.VMEM((64,), jnp.int32),
        pltpu.VMEM_SHARED((256,), jnp.int32),
        pltpu.SemaphoreType.DMA,
    ],
)
def _route_sc(scores_hbm, f_hbm, reci_hbm, recw_hbm, fs_hbm, te_hbm,
              sc_v, f_v, ee_v, ww_v, reci_v, loc_pos0, loc_pos1, hist_v,
              allhist_v, te_v, shist, sem):
    ii = lax.iota(jnp.int32, 16)
    lane0 = ii == 0
    lane1 = ii == 1
    wid = lax.axis_index("s")
    base = wid * TPA
    pltpu.sync_copy(scores_hbm.at[pl.ds(base * 16, TPA * 16)], sc_v)
    pltpu.sync_copy(f_hbm.at[pl.ds(base, TPA)], f_v)

    def p1(i, hist):
        row = sc_v[pl.ds(16 * i, 16)]
        m1 = jnp.max(row)
        e0 = plsc.all_reduce_ffs(row == m1)
        onehot0 = ii == e0
        row2 = jnp.where(onehot0, -jnp.inf, row)
        m2 = jnp.max(row2)
        e1 = plsc.all_reduce_ffs(row2 == m2)
        onehot1 = ii == e1
        ex = jnp.exp(jnp.full((16,), 1.0, jnp.float32) * (m2 - m1))
        wtop = 1.0 / (1.0 + ex)
        wsec = ex / (1.0 + ex)
        eerow = jnp.where(lane0, e0, jnp.where(lane1, e1, 0)).astype(jnp.int32)
        wwrow = jnp.where(lane0, wtop,
                          jnp.where(lane1, wsec, 0.0)).astype(jnp.float32)
        ee_v[pl.ds(16 * i, 16)] = eerow
        ww_v[pl.ds(16 * i, 16)] = wwrow
        return hist + onehot0.astype(jnp.int32) + onehot1.astype(jnp.int32)

    hist = lax.fori_loop(0, TPA, p1, jnp.zeros((16,), jnp.int32))
    hist_v[...] = hist
    pltpu.sync_copy(hist_v, shist.at[pl.ds(wid * 16, 16)])
    plsc.subcore_barrier()
    pltpu.sync_copy(shist, allhist_v)

    counts = jnp.zeros((16,), jnp.int32)
    mypre = jnp.zeros((16,), jnp.int32)
    for r in range(16):
        hrow = allhist_v[pl.ds(16 * r, 16)]
        counts = counts + hrow
        mypre = mypre + jnp.where(r < wid, hrow, 0)
    pcounts = ((counts + (TB - 1)) >> 7) << 7
    poff = plsc.cumsum(pcounts) - pcounts
    pend = poff + pcounts

    @pl.when(wid == 0)
    def _te():
        for j in range(4):
            if j < 3:
                tvec = (ii + 16 * j) * TB
                acc = jnp.zeros((16,), jnp.int32)
                for e_ in range(16):
                    pe = jnp.sum(jnp.where(ii == e_, pend, 0))
                    acc = acc + (tvec >= pe).astype(jnp.int32)
                acc = jnp.minimum(acc, E - 1)
            else:
                acc = jnp.full((16,), E - 1, jnp.int32)
            plsc.store_scatter(te_v, [ii + 16 * j], acc)
        pltpu.sync_copy(te_v, te_hbm)

    def p2(i, cursor):
        eerow = ee_v[pl.ds(16 * i, 16)]
        e0 = jnp.sum(jnp.where(lane0, eerow, 0))
        e1 = jnp.sum(jnp.where(lane1, eerow, 0))
        onehot0 = ii == e0
        pos0 = jnp.sum(jnp.where(onehot0, cursor, 0))
        cursor = cursor + onehot0.astype(jnp.int32)
        onehot1 = ii == e1
        pos1 = jnp.sum(jnp.where(onehot1, cursor, 0))
        cursor = cursor + onehot1.astype(jnp.int32)
        recrow = jnp.where(lane0, pos0, jnp.where(lane1, pos1, 0))
        reci_v[pl.ds(16 * i, 16)] = recrow.astype(jnp.int32)
        plsc.store_scatter(loc_pos0, [ii * 0 + i], ii * 0 + pos0, mask=lane0)
        plsc.store_scatter(loc_pos1, [ii * 0 + i], ii * 0 + pos1, mask=lane0)
        return cursor

    lax.fori_loop(0, TPA, p2, poff + mypre)
    pltpu.sync_copy(reci_v, reci_hbm.at[pl.ds(base * 16, TPA * 16)])
    pltpu.sync_copy(ww_v, recw_hbm.at[pl.ds(base * 16, TPA * 16)])
    pltpu.async_copy(f_v, fs_hbm.at[loc_pos0], sem).wait()
    pltpu.async_copy(f_v, fs_hbm.at[loc_pos1], sem).wait()


@functools.partial(
    pl.kernel, mesh=_mesh_c,
    compiler_params=pltpu.CompilerParams(needs_layout_passes=False),
    out_type=jax.ShapeDtypeStruct((S, D), jnp.float32),
    scratch_types=[
        pltpu.VMEM((TPC * 16,), jnp.int32),
        pltpu.VMEM((TPC * 16,), jnp.float32),
        pltpu.VMEM((CCH, D), jnp.float32),
        pltpu.VMEM((2 * CCH, D), jnp.float32),
        pltpu.VMEM((2 * CCH,), jnp.int32),
        pltpu.SemaphoreType.DMA,
    ],
)
def _combine_sc(reci_hbm, recw_hbm, x1_hbm, y_hbm, out_hbm,
                reci_v, recw_v, x1_v, y_v, idx_v, sem):
    ii = lax.iota(jnp.int32, 16)
    lane0 = ii == 0
    lane1 = ii == 1
    wid = lax.axis_index("s") * 2 + lax.axis_index("c")
    tbase = wid * TPC
    pltpu.sync_copy(reci_hbm.at[pl.ds(tbase * 16, TPC * 16)], reci_v)
    pltpu.sync_copy(recw_hbm.at[pl.ds(tbase * 16, TPC * 16)], recw_v)
    for h in range(TPC // CCH):
        cb = h * CCH
        pltpu.sync_copy(x1_hbm.at[pl.ds(tbase + cb, CCH)], x1_v)

        def build(i, z):
            rec = reci_v[pl.ds(16 * (cb + i), 16)]
            pos0 = jnp.sum(jnp.where(lane0, rec, 0))
            pos1 = jnp.sum(jnp.where(lane1, rec, 0))
            plsc.store_scatter(idx_v, [ii * 0 + 2 * i], ii * 0 + pos0,
                               mask=lane0)
            plsc.store_scatter(idx_v, [ii * 0 + 2 * i + 1], ii * 0 + pos1,
                               mask=lane0)
            return z

        lax.fori_loop(0, CCH, build, 0)
        pltpu.async_copy(y_hbm.at[idx_v], y_v, sem).wait()

        def comb(i, z):
            ww = recw_v[pl.ds(16 * (cb + i), 16)]
            w0 = jnp.sum(jnp.where(lane0, ww, 0.0))
            w1 = jnp.sum(jnp.where(lane1, ww, 0.0))
            for c in range(D // 16):
                sl = pl.ds(16 * c, 16)
                x1_v[i, sl] = (x1_v[i, sl] + w0 * y_v[2 * i, sl]
                               + w1 * y_v[2 * i + 1, sl])
            return z

        lax.fori_loop(0, CCH, comb, 0)
        pltpu.sync_copy(x1_v, out_hbm.at[pl.ds(tbase + cb, CCH)])


def kernel(x, ln1_w, ln1_b, in_proj_w, in_proj_b, out_proj_w, out_proj_b,
           ln2_w, ln2_b, gate_w, gate_b, W1, B1, W2, B2):
    x2 = x.reshape(S, D)
    qkv = _qkv_proj(x2, ln1_w, ln1_b, in_proj_w, in_proj_b)
    o = _attention(qkv)
    x1, f, scores = _post_attn(o, x2, out_proj_w, out_proj_b,
                               ln2_w, ln2_b, gate_w, gate_b)
    reci, recw, fsorted, te = _route_sc(scores.reshape(S * 16), f)
    y = _grouped_ffn(te, fsorted, W1, B1, W2, B2)
    out = _combine_sc(reci, recw, x1, y)
    return out.reshape(1, S, D)


# speedup vs baseline: 3.5587x; 1.0859x over previous
import functools

import jax
import jax.numpy as jnp
from jax import lax
from jax.experimental import pallas as pl
from jax.experimental.pallas import tpu as pltpu
from jax.experimental.pallas import tpu_sc as plsc

S, D, H, E, TOPK = 2048, 768, 12, 16, 2
DFF = 4 * D
DH = D // H
TB = 128
PAD = S * TOPK + E * TB
NT = PAD // TB


def _k1_body(x_ref, lw_ref, lb_ref, w_ref, b_ref, qkv_ref):
    x = x_ref[...]
    mu = jnp.mean(x, axis=-1, keepdims=True)
    var = jnp.mean((x - mu) ** 2, axis=-1, keepdims=True)
    h = (x - mu) / jnp.sqrt(var + 1e-5) * lw_ref[...] + lb_ref[...]
    qkv_ref[...] = jax.lax.dot_general(
        h, w_ref[...], (((1,), (1,)), ((), ())),
        preferred_element_type=jnp.float32) + b_ref[...]


def _qkv_proj(x, ln1_w, ln1_b, w, b):
    bs = 512
    return pl.pallas_call(
        _k1_body,
        grid=(S // bs,),
        in_specs=[
            pl.BlockSpec((bs, D), lambda i: (i, 0)),
            pl.BlockSpec((1, D), lambda i: (0, 0)),
            pl.BlockSpec((1, D), lambda i: (0, 0)),
            pl.BlockSpec((3 * D, D), lambda i: (0, 0)),
            pl.BlockSpec((1, 3 * D), lambda i: (0, 0)),
        ],
        out_specs=pl.BlockSpec((bs, 3 * D), lambda i: (i, 0)),
        out_shape=jax.ShapeDtypeStruct((S, 3 * D), jnp.float32),
    )(x, ln1_w.reshape(1, D), ln1_b.reshape(1, D), w, b.reshape(1, 3 * D))


def _k2_body(q_ref, k_ref, v_ref, o_ref):
    scale = 1.0 / (DH ** 0.5)
    outs = []
    for sub in range(2):
        q = q_ref[:, sub * DH:(sub + 1) * DH]
        k = k_ref[:, sub * DH:(sub + 1) * DH]
        v = v_ref[:, sub * DH:(sub + 1) * DH]
        s = jax.lax.dot_general(q, k, (((1,), (1,)), ((), ())),
                                                        preferred_element_type=jnp.float32) * scale
        m = jnp.max(s, axis=-1, keepdims=True)
        p = jnp.exp(s - m)
        l = jnp.sum(p, axis=-1, keepdims=True)
        o = jax.lax.dot_general(p, v, (((1,), (0,)), ((), ())),
                                                        preferred_element_type=jnp.float32)
        outs.append(o / l)
    o_ref[...] = jnp.concatenate(outs, axis=1)


def _attention(qkv):
    qb = 512
    return pl.pallas_call(
        _k2_body,
        grid=(H // 2, S // qb),
        in_specs=[
            pl.BlockSpec((qb, 2 * DH), lambda p, i: (i, p)),
            pl.BlockSpec((S, 2 * DH), lambda p, i: (0, 6 + p)),
            pl.BlockSpec((S, 2 * DH), lambda p, i: (0, 12 + p)),
        ],
        out_specs=pl.BlockSpec((qb, 2 * DH), lambda p, i: (i, p)),
        out_shape=jax.ShapeDtypeStruct((S, D), jnp.float32),
    )(qkv, qkv, qkv)


def _k3_body(o_ref, x_ref, w_ref, b_ref, lw_ref, lb_ref, gw_ref, gb_ref,
             x1_ref, f_ref, sc_ref):
    x1 = x_ref[...] + jax.lax.dot_general(
        o_ref[...], w_ref[...], (((1,), (1,)), ((), ())),
        preferred_element_type=jnp.float32) + b_ref[...]
    x1_ref[...] = x1
    mu = jnp.mean(x1, axis=-1, keepdims=True)
    var = jnp.mean((x1 - mu) ** 2, axis=-1, keepdims=True)
    f = (x1 - mu) / jnp.sqrt(var + 1e-5) * lw_ref[...] + lb_ref[...]
    f_ref[...] = f
    sc_ref[...] = jax.lax.dot_general(
        f, gw_ref[...], (((1,), (1,)), ((), ())),
        preferred_element_type=jnp.float32) + gb_ref[...]


def _post_attn(o, x, w, b, ln2_w, ln2_b, gate_w, gate_b):
    bs = 256
    return pl.pallas_call(
        _k3_body,
        grid=(S // bs,),
        in_specs=[
            pl.BlockSpec((bs, D), lambda i: (i, 0)),
            pl.BlockSpec((bs, D), lambda i: (i, 0)),
            pl.BlockSpec((D, D), lambda i: (0, 0)),
            pl.BlockSpec((1, D), lambda i: (0, 0)),
            pl.BlockSpec((1, D), lambda i: (0, 0)),
            pl.BlockSpec((1, D), lambda i: (0, 0)),
            pl.BlockSpec((E, D), lambda i: (0, 0)),
            pl.BlockSpec((1, E), lambda i: (0, 0)),
        ],
        out_specs=[
            pl.BlockSpec((bs, D), lambda i: (i, 0)),
            pl.BlockSpec((bs, D), lambda i: (i, 0)),
            pl.BlockSpec((bs, E), lambda i: (i, 0)),
        ],
        out_shape=[
            jax.ShapeDtypeStruct((S, D), jnp.float32),
            jax.ShapeDtypeStruct((S, D), jnp.float32),
            jax.ShapeDtypeStruct((S, E), jnp.float32),
        ],
    )(o, x, w, b.reshape(1, D), ln2_w.reshape(1, D), ln2_b.reshape(1, D),
      gate_w, gate_b.reshape(1, E))


def _k6_body(te_ref, nv_ref, f_ref, w1_ref, b1_ref, w2_ref, b2_ref, y_ref,
             w1b, w2b):
    t = pl.program_id(0)

    @pl.when(t < nv_ref[0])
    def _go():
        changed = (t == 0) | (te_ref[t] != te_ref[jnp.maximum(t - 1, 0)])

        @pl.when(changed)
        def _cast():
            w1b[...] = w1_ref[0].astype(jnp.bfloat16)
            w2b[...] = w2_ref[0].astype(jnp.bfloat16)

        a = jax.lax.dot_general(
            f_ref[...].astype(jnp.bfloat16), w1b[...], (((1,), (1,)), ((), ())),
            preferred_element_type=jnp.float32) + b1_ref[0]
        g = 0.5 * a * (1.0 + jax.lax.erf(a * (2.0 ** -0.5)))
        y = jax.lax.dot_general(
            g.astype(jnp.bfloat16), w2b[...], (((1,), (1,)), ((), ())),
            preferred_element_type=jnp.float32) + b2_ref[0]
        y_ref[...] = y


def _grouped_ffn(tile_expert, nvalid, fsorted, W1, B1, W2, B2):
    grid_spec = pltpu.PrefetchScalarGridSpec(
        num_scalar_prefetch=2,
        grid=(NT,),
        in_specs=[
            pl.BlockSpec((TB, D),
                         lambda t, te, nv: (jnp.minimum(t, nv[0] - 1), 0)),
            pl.BlockSpec((1, DFF, D), lambda t, te, nv: (te[t], 0, 0)),
            pl.BlockSpec((1, 1, DFF), lambda t, te, nv: (te[t], 0, 0)),
            pl.BlockSpec((1, D, DFF), lambda t, te, nv: (te[t], 0, 0)),
            pl.BlockSpec((1, 1, D), lambda t, te, nv: (te[t], 0, 0)),
        ],
        out_specs=pl.BlockSpec((TB, D), lambda t, te, nv: (t, 0)),
        scratch_shapes=[
            pltpu.VMEM((DFF, D), jnp.bfloat16),
            pltpu.VMEM((D, DFF), jnp.bfloat16),
        ],
    )
    return pl.pallas_call(
        _k6_body,
        grid_spec=grid_spec,
        out_shape=jax.ShapeDtypeStruct((PAD, D), jnp.float32),
    )(tile_expert, nvalid, fsorted, W1, B1.reshape(E, 1, DFF), W2,
      B2.reshape(E, 1, D))


TPA = S // 16
TPC = S // 32
CCH = 32

_mesh_a = plsc.VectorSubcoreMesh(core_axis_name="c", subcore_axis_name="s",
                                 num_cores=1)
_mesh_c = plsc.VectorSubcoreMesh(core_axis_name="c", subcore_axis_name="s")


@functools.partial(
    pl.kernel, mesh=_mesh_a,
    compiler_params=pltpu.CompilerParams(needs_layout_passes=False),
    out_type=[
        jax.ShapeDtypeStruct((S * 16,), jnp.int32),
        jax.ShapeDtypeStruct((S * 16,), jnp.float32),
        jax.ShapeDtypeStruct((PAD, D), jnp.float32),
        jax.ShapeDtypeStruct((64,), jnp.int32),
        jax.ShapeDtypeStruct((8,), jnp.int32),
    ],
    scratch_types=[
        pltpu.VMEM((TPA * 16,), jnp.float32),
        pltpu.VMEM((TPA, D), jnp.float32),
        pltpu.VMEM((TPA * 16,), jnp.int32),
        pltpu.VMEM((TPA * 16,), jnp.float32),
        pltpu.VMEM((TPA * 16,), jnp.int32),
        pltpu.VMEM((TPA,), jnp.int32),
        pltpu.VMEM((TPA,), jnp.int32),
        pltpu.VMEM((16,), jnp.int32),
        pltpu.VMEM((256,), jnp.int32),
        pltpu.VMEM((64,), jnp.int32),
        pltpu.VMEM((16,), jnp.int32),
        pltpu.VMEM_SHARED((256,), jnp.int32),
        pltpu.SemaphoreType.DMA,
    ],
)
def _route_sc(scores_hbm, f_hbm, reci_hbm, recw_hbm, fs_hbm, te_hbm, nv_hbm,
              sc_v, f_v, ee_v, ww_v, reci_v, loc_pos0, loc_pos1, hist_v,
              allhist_v, te_v, nv_v, shist, sem):
    ii = lax.iota(jnp.int32, 16)
    lane0 = ii == 0
    lane1 = ii == 1
    wid = lax.axis_index("s")
    base = wid * TPA
    pltpu.sync_copy(scores_hbm.at[pl.ds(base * 16, TPA * 16)], sc_v)
    pltpu.sync_copy(f_hbm.at[pl.ds(base, TPA)], f_v)

    def p1(i, hist):
        row = sc_v[pl.ds(16 * i, 16)]
        m1 = jnp.max(row)
        e0 = plsc.all_reduce_ffs(row == m1)
        onehot0 = ii == e0
        row2 = jnp.where(onehot0, -jnp.inf, row)
        m2 = jnp.max(row2)
        e1 = plsc.all_reduce_ffs(row2 == m2)
        onehot1 = ii == e1
        ex = jnp.exp(jnp.full((16,), 1.0, jnp.float32) * (m2 - m1))
        wtop = 1.0 / (1.0 + ex)
        wsec = ex / (1.0 + ex)
        eerow = jnp.where(lane0, e0, jnp.where(lane1, e1, 0)).astype(jnp.int32)
        wwrow = jnp.where(lane0, wtop,
                          jnp.where(lane1, wsec, 0.0)).astype(jnp.float32)
        ee_v[pl.ds(16 * i, 16)] = eerow
        ww_v[pl.ds(16 * i, 16)] = wwrow
        return hist + onehot0.astype(jnp.int32) + onehot1.astype(jnp.int32)

    hist = lax.fori_loop(0, TPA, p1, jnp.zeros((16,), jnp.int32))
    hist_v[...] = hist
    pltpu.sync_copy(hist_v, shist.at[pl.ds(wid * 16, 16)])
    plsc.subcore_barrier()
    pltpu.sync_copy(shist, allhist_v)

    counts = jnp.zeros((16,), jnp.int32)
    mypre = jnp.zeros((16,), jnp.int32)
    for r in range(16):
        hrow = allhist_v[pl.ds(16 * r, 16)]
        counts = counts + hrow
        mypre = mypre + jnp.where(r < wid, hrow, 0)
    pcounts = ((counts + (TB - 1)) >> 7) << 7
    poff = plsc.cumsum(pcounts) - pcounts
    pend = poff + pcounts

    @pl.when(wid == 0)
    def _te():
        for j in range(4):
            if j < 3:
                tvec = (ii + 16 * j) * TB
                acc = jnp.zeros((16,), jnp.int32)
                for e_ in range(16):
                    pe = jnp.sum(jnp.where(ii == e_, pend, 0))
                    acc = acc + (tvec >= pe).astype(jnp.int32)
                acc = jnp.minimum(acc, E - 1)
            else:
                acc = jnp.full((16,), E - 1, jnp.int32)
            plsc.store_scatter(te_v, [ii + 16 * j], acc)
        pltpu.sync_copy(te_v, te_hbm)
        total = jnp.sum(pcounts)
        nv_v[...] = ii * 0 + total // TB
        pltpu.sync_copy(nv_v.at[pl.ds(0, 8)], nv_hbm)

    def p2(i, cursor):
        eerow = ee_v[pl.ds(16 * i, 16)]
        e0 = jnp.sum(jnp.where(lane0, eerow, 0))
        e1 = jnp.sum(jnp.where(lane1, eerow, 0))
        onehot0 = ii == e0
        pos0 = jnp.sum(jnp.where(onehot0, cursor, 0))
        cursor = cursor + onehot0.astype(jnp.int32)
        onehot1 = ii == e1
        pos1 = jnp.sum(jnp.where(onehot1, cursor, 0))
        cursor = cursor + onehot1.astype(jnp.int32)
        recrow = jnp.where(lane0, pos0, jnp.where(lane1, pos1, 0))
        reci_v[pl.ds(16 * i, 16)] = recrow.astype(jnp.int32)
        plsc.store_scatter(loc_pos0, [ii * 0 + i], ii * 0 + pos0, mask=lane0)
        plsc.store_scatter(loc_pos1, [ii * 0 + i], ii * 0 + pos1, mask=lane0)
        return cursor

    lax.fori_loop(0, TPA, p2, poff + mypre)
    pltpu.sync_copy(reci_v, reci_hbm.at[pl.ds(base * 16, TPA * 16)])
    pltpu.sync_copy(ww_v, recw_hbm.at[pl.ds(base * 16, TPA * 16)])
    pltpu.async_copy(f_v, fs_hbm.at[loc_pos0], sem).wait()
    pltpu.async_copy(f_v, fs_hbm.at[loc_pos1], sem).wait()


@functools.partial(
    pl.kernel, mesh=_mesh_c,
    compiler_params=pltpu.CompilerParams(needs_layout_passes=False),
    out_type=jax.ShapeDtypeStruct((S, D), jnp.float32),
    scratch_types=[
        pltpu.VMEM((TPC * 16,), jnp.int32),
        pltpu.VMEM((TPC * 16,), jnp.float32),
        pltpu.VMEM((CCH, D), jnp.float32),
        pltpu.VMEM((2 * CCH, D), jnp.float32),
        pltpu.VMEM((2 * CCH,), jnp.int32),
        pltpu.SemaphoreType.DMA,
    ],
)
def _combine_sc(reci_hbm, recw_hbm, x1_hbm, y_hbm, out_hbm,
                reci_v, recw_v, x1_v, y_v, idx_v, sem):
    ii = lax.iota(jnp.int32, 16)
    lane0 = ii == 0
    lane1 = ii == 1
    wid = lax.axis_index("s") * 2 + lax.axis_index("c")
    tbase = wid * TPC
    pltpu.sync_copy(reci_hbm.at[pl.ds(tbase * 16, TPC * 16)], reci_v)
    pltpu.sync_copy(recw_hbm.at[pl.ds(tbase * 16, TPC * 16)], recw_v)
    for h in range(TPC // CCH):
        cb = h * CCH
        pltpu.sync_copy(x1_hbm.at[pl.ds(tbase + cb, CCH)], x1_v)

        def build(i, z):
            rec = reci_v[pl.ds(16 * (cb + i), 16)]
            pos0 = jnp.sum(jnp.where(lane0, rec, 0))
            pos1 = jnp.sum(jnp.where(lane1, rec, 0))
            plsc.store_scatter(idx_v, [ii * 0 + 2 * i], ii * 0 + pos0,
                               mask=lane0)
            plsc.store_scatter(idx_v, [ii * 0 + 2 * i + 1], ii * 0 + pos1,
                               mask=lane0)
            return z

        lax.fori_loop(0, CCH, build, 0)
        pltpu.async_copy(y_hbm.at[idx_v], y_v, sem).wait()

        def comb(i, z):
            ww = recw_v[pl.ds(16 * (cb + i), 16)]
            w0 = jnp.sum(jnp.where(lane0, ww, 0.0))
            w1 = jnp.sum(jnp.where(lane1, ww, 0.0))
            for c in range(D // 16):
                sl = pl.ds(16 * c, 16)
                x1_v[i, sl] = (x1_v[i, sl] + w0 * y_v[2 * i, sl]
                               + w1 * y_v[2 * i + 1, sl])
            return z

        lax.fori_loop(0, CCH, comb, 0)
        pltpu.sync_copy(x1_v, out_hbm.at[pl.ds(tbase + cb, CCH)])


def kernel(x, ln1_w, ln1_b, in_proj_w, in_proj_b, out_proj_w, out_proj_b,
           ln2_w, ln2_b, gate_w, gate_b, W1, B1, W2, B2):
    x2 = x.reshape(S, D)
    qkv = _qkv_proj(x2, ln1_w, ln1_b, in_proj_w, in_proj_b)
    o = _attention(qkv)
    x1, f, scores = _post_attn(o, x2, out_proj_w, out_proj_b,
                               ln2_w, ln2_b, gate_w, gate_b)
    reci, recw, fsorted, te, nv = _route_sc(scores.reshape(S * 16), f)
    y = _grouped_ffn(te, nv, fsorted, W1, B1, W2, B2)
    out = _combine_sc(reci, recw, x1, y)
    return out.reshape(1, S, D)
